# Initial kernel scaffold; baseline (speedup 1.0000x reference)
import jax
import jax.numpy as jnp
from jax.experimental import pallas as pl


def kernel(x, edge_index, edge_attr, W1, b1, gamma, beta, W2, b2):
    raise NotImplementedError("write your pallas kernel here")


# speedup vs baseline: 1.7460x; 1.7460x over previous
import dataclasses
import functools

import jax
import jax.numpy as jnp
from jax import lax
from jax.experimental import pallas as pl
from jax.experimental.pallas import tpu as pltpu
from jax.experimental.pallas import tpu_sc as plsc

N = 10000
E = 320000
D = 128
DE = 16
NC, NS, L = 2, 16, 16
NW = NC * NS
EPW = E // NW
CH = 80
NCHUNK = EPW // CH
NST = 10
RPT = N // NST

_mesh = plsc.VectorSubcoreMesh(
    core_axis_name="c", subcore_axis_name="s", num_cores=NC, num_subcores=NS
)

_sc_params = pltpu.CompilerParams()
if "needs_layout_passes" in pltpu.CompilerParams.__dataclass_fields__:
    _sc_params = dataclasses.replace(_sc_params, needs_layout_passes=False)


def _node_proj_body(x_ref, w_ref, xa_ref, xb_ref):
    xab = jnp.dot(x_ref[...], w_ref[...], preferred_element_type=jnp.float32)
    xa_ref[...] = xab[:, :D]
    xb_ref[...] = xab[:, D:]


def _node_proj(x, w1abt):
    return pl.pallas_call(
        _node_proj_body,
        out_shape=(
            jax.ShapeDtypeStruct((N, D), jnp.float32),
            jax.ShapeDtypeStruct((N, D), jnp.float32),
        ),
    )(x, w1abt)


def _edge_proj_body(ea_ref, w_ref, b_ref, out_ref):
    out_ref[...] = (
        jnp.dot(ea_ref[...], w_ref[...], preferred_element_type=jnp.float32)
        + b_ref[...]
    )


def _edge_proj(edge_attr, w1ct, b1):
    BE = 10000
    return pl.pallas_call(
        _edge_proj_body,
        out_shape=jax.ShapeDtypeStruct((E, D), jnp.float32),
        grid=(E // BE,),
        in_specs=[
            pl.BlockSpec((BE, DE), lambda i: (i, 0)),
            pl.BlockSpec((DE, D), lambda i: (0, 0)),
            pl.BlockSpec((1, D), lambda i: (0, 0)),
        ],
        out_specs=pl.BlockSpec((BE, D), lambda i: (i, 0)),
    )(edge_attr, w1ct, b1.reshape(1, D))


@functools.partial(
    pl.kernel,
    out_type=(
        jax.ShapeDtypeStruct((E, D), jnp.float32),
        jax.ShapeDtypeStruct((NW, 2, D), jnp.float32),
        jax.ShapeDtypeStruct((NW, N), jnp.float32),
    ),
    mesh=_mesh,
    scratch_types=[
        pltpu.VMEM((CH,), jnp.int32),
        pltpu.VMEM((CH,), jnp.int32),
        pltpu.VMEM((CH, D), jnp.float32),
        pltpu.VMEM((CH, D), jnp.float32),
        pltpu.VMEM((CH, D), jnp.float32),
        pltpu.VMEM((2, D), jnp.float32),
        pltpu.VMEM((N,), jnp.float32),
        pltpu.SemaphoreType.DMA,
        pltpu.SemaphoreType.DMA,
        pltpu.SemaphoreType.DMA,
    ],
    compiler_params=_sc_params,
)
def _sc_pass1(xa_hbm, xb_hbm, ea_hbm, row_hbm, col_hbm,
              h_hbm, stats_hbm, cnt_hbm,
              rowv, colv, bufa, bufb, bufe, stats, counts,
              sema, semb, seme):
    cid = lax.axis_index("c")
    sid = lax.axis_index("s")
    wid = sid * NC + cid
    base = wid * EPW

    zero16 = jnp.zeros((L,), jnp.float32)
    ones16 = jnp.full((L,), 1.0, jnp.float32)

    @pl.loop(0, N // L)
    def _(i):
        counts[pl.ds(i * L, L)] = zero16

    @pl.loop(0, D // L)
    def _(k):
        stats[0, pl.ds(k * L, L)] = zero16
        stats[1, pl.ds(k * L, L)] = zero16

    @pl.loop(0, NCHUNK)
    def _(t):
        off = base + t * CH
        pltpu.sync_copy(row_hbm.at[pl.ds(off, CH)], rowv)
        pltpu.sync_copy(col_hbm.at[pl.ds(off, CH)], colv)
        cpa = pltpu.async_copy(xa_hbm.at[rowv], bufa, sema)
        cpb = pltpu.async_copy(xb_hbm.at[colv], bufb, semb)
        cpe = pltpu.async_copy(ea_hbm.at[pl.ds(off, CH)], bufe, seme)

        @pl.loop(0, CH // L)
        def _(j):
            idxv = rowv[pl.ds(j * L, L)]
            plsc.addupdate_scatter(counts, [idxv], ones16)

        cpa.wait()
        cpb.wait()
        cpe.wait()

        @pl.loop(0, CH)
        def _(e):
            @pl.loop(0, D // L)
            def _(k):
                sl = pl.ds(k * L, L)
                h = bufa[e, sl] + bufb[e, sl] + bufe[e, sl]
                bufe[e, sl] = h
                plsc.addupdate(stats.at[0, sl], h)
                plsc.addupdate(stats.at[1, sl], h * h)

        pltpu.sync_copy(bufe, h_hbm.at[pl.ds(off, CH)])

    pltpu.sync_copy(stats, stats_hbm.at[wid])
    pltpu.sync_copy(counts, cnt_hbm.at[wid])


def _stats_body(stats_ref, g_ref, b_ref, ss_ref):
    s = jnp.sum(stats_ref[...], axis=0)
    mean = s[0:1, :] * (1.0 / E)
    ex2 = s[1:2, :] * (1.0 / E)
    var = ex2 - mean * mean
    inv = lax.rsqrt(var + 1e-5)
    scale = g_ref[...] * inv
    shift = b_ref[...] - mean * scale
    ss_ref[...] = jnp.concatenate([scale, shift], axis=0)


def _stats_reduce(stats, gamma, beta):
    return pl.pallas_call(
        _stats_body,
        out_shape=jax.ShapeDtypeStruct((2, D), jnp.float32),
    )(stats, gamma.reshape(1, D), beta.reshape(1, D))


@functools.partial(
    pl.kernel,
    out_type=jax.ShapeDtypeStruct((NC, N, D), jnp.float32),
    mesh=_mesh,
    scratch_types=[
        pltpu.VMEM((1, CH), jnp.int32),
        pltpu.VMEM((CH, D), jnp.float32),
        pltpu.VMEM((2, D), jnp.float32),
        pltpu.VMEM_SHARED((N, D), jnp.float32),
        pltpu.SemaphoreType.DMA,
    ],
    compiler_params=_sc_params,
)
def _sc_pass2(h_hbm, row_hbm, ss_hbm, zeros_hbm,
              s_hbm,
              rowv, buf, ss, s_sh, sem):
    cid = lax.axis_index("c")
    sid = lax.axis_index("s")
    wid = sid * NC + cid
    base = wid * EPW

    pltpu.sync_copy(ss_hbm, ss)

    @pl.when(sid < NST)
    def _():
        pltpu.sync_copy(zeros_hbm, s_sh.at[pl.ds(sid * RPT, RPT)])

    plsc.subcore_barrier()

    @pl.loop(0, NCHUNK)
    def _(t):
        off = base + t * CH
        pltpu.sync_copy(row_hbm.at[pl.ds(off, CH)], rowv.at[0])
        pltpu.sync_copy(h_hbm.at[pl.ds(off, CH)], buf)

        @pl.loop(0, CH)
        def _(e):
            @pl.loop(0, D // L)
            def _(k):
                sl = pl.ds(k * L, L)
                y = buf[e, sl] * ss[0, sl] + ss[1, sl]
                buf[e, sl] = jnp.maximum(y, 0.0)

        pltpu.sync_copy(buf, s_sh.at[rowv.at[0]], add=True)

    plsc.subcore_barrier()

    @pl.when(sid < NST)
    def _():
        pltpu.sync_copy(
            s_sh.at[pl.ds(sid * RPT, RPT)],
            s_hbm.at[cid].at[pl.ds(sid * RPT, RPT)],
        )


def _final_body(s_ref, cnt_ref, x_ref, w_ref, b_ref, out_ref):
    s = s_ref[0] + s_ref[1]
    cnt = jnp.sum(cnt_ref[...], axis=0)[:, None]
    m = jnp.dot(s, w_ref[...], preferred_element_type=jnp.float32)
    out_ref[...] = (m + cnt * b_ref[...]) / (cnt + 1.0) + x_ref[...]


def _final(s_parts, cnts, x, w2t, b2):
    return pl.pallas_call(
        _final_body,
        out_shape=jax.ShapeDtypeStruct((N, D), jnp.float32),
    )(s_parts, cnts, x, w2t, b2.reshape(1, D))


def kernel(x, edge_index, edge_attr, W1, b1, gamma, beta, W2, b2):
    row = edge_index[0].astype(jnp.int32)
    col = edge_index[1].astype(jnp.int32)
    w1abt = jnp.concatenate([W1[:, :D].T, W1[:, D : 2 * D].T], axis=1)
    w1ct = W1[:, 2 * D :].T
    w2t = W2.T

    xa, xb = _node_proj(x, w1abt)
    ea = _edge_proj(edge_attr, w1ct, b1)
    h, stats, cnts = _sc_pass1(xa, xb, ea, row, col)
    ss = _stats_reduce(stats, gamma, beta)
    zeros = jnp.zeros((RPT, D), jnp.float32)
    s_parts = _sc_pass2(h, row, ss, zeros)
    return _final(s_parts, cnts, x, w2t, b2)


# speedup vs baseline: 5.2045x; 2.9808x over previous
import dataclasses
import functools

import jax
import jax.numpy as jnp
from jax import lax
from jax.experimental import pallas as pl
from jax.experimental.pallas import tpu as pltpu
from jax.experimental.pallas import tpu_sc as plsc

N = 10000
E = 320000
D = 128
DE = 16
NC, NS, L = 2, 16, 16
NW = NC * NS
EPW = E // NW
CH = 80
NCHUNK = EPW // CH
NST = 10
RPT = N // NST

_mesh = plsc.VectorSubcoreMesh(
    core_axis_name="c", subcore_axis_name="s", num_cores=NC, num_subcores=NS
)

_sc_params = pltpu.CompilerParams()
if "needs_layout_passes" in pltpu.CompilerParams.__dataclass_fields__:
    _sc_params = dataclasses.replace(_sc_params, needs_layout_passes=False)


def _node_proj_body(x_ref, w_ref, xa_ref, xb_ref):
    xab = jnp.dot(x_ref[...], w_ref[...], preferred_element_type=jnp.float32)
    xa_ref[...] = xab[:, :D]
    xb_ref[...] = xab[:, D:]


def _node_proj(x, w1abt):
    return pl.pallas_call(
        _node_proj_body,
        out_shape=(
            jax.ShapeDtypeStruct((N, D), jnp.float32),
            jax.ShapeDtypeStruct((N, D), jnp.float32),
        ),
    )(x, w1abt)


def _edge_proj_body(ea_ref, w_ref, b_ref, out_ref):
    out_ref[...] = (
        jnp.dot(ea_ref[...], w_ref[...], preferred_element_type=jnp.float32)
        + b_ref[...]
    )


def _edge_proj(edge_attr, w1ct, b1):
    BE = 10000
    return pl.pallas_call(
        _edge_proj_body,
        out_shape=jax.ShapeDtypeStruct((E, D), jnp.float32),
        grid=(E // BE,),
        in_specs=[
            pl.BlockSpec((BE, DE), lambda i: (i, 0)),
            pl.BlockSpec((DE, D), lambda i: (0, 0)),
            pl.BlockSpec((1, D), lambda i: (0, 0)),
        ],
        out_specs=pl.BlockSpec((BE, D), lambda i: (i, 0)),
    )(edge_attr, w1ct, b1.reshape(1, D))


@functools.partial(
    pl.kernel,
    out_type=(
        jax.ShapeDtypeStruct((E, D), jnp.float32),
        jax.ShapeDtypeStruct((NW, 2, D), jnp.float32),
        jax.ShapeDtypeStruct((NW, N), jnp.float32),
    ),
    mesh=_mesh,
    scratch_types=[
        pltpu.VMEM((NCHUNK, CH), jnp.int32),
        pltpu.VMEM((NCHUNK, CH), jnp.int32),
        pltpu.VMEM((2, CH, D), jnp.float32),
        pltpu.VMEM((2, CH, D), jnp.float32),
        pltpu.VMEM((2, CH, D), jnp.float32),
        pltpu.VMEM((2, D), jnp.float32),
        pltpu.VMEM((N,), jnp.float32),
        pltpu.SemaphoreType.DMA,
        pltpu.SemaphoreType.DMA,
        pltpu.SemaphoreType.DMA,
        pltpu.SemaphoreType.DMA,
        pltpu.SemaphoreType.DMA,
        pltpu.SemaphoreType.DMA,
    ],
    compiler_params=_sc_params,
)
def _sc_pass1(xa_hbm, xb_hbm, ea_hbm, row_hbm, col_hbm,
              h_hbm, stats_hbm, cnt_hbm,
              rowsb, colsb, bufa, bufb, bufe, stats, counts,
              sa0, sa1, sb0, sb1, se0, se1):
    cid = lax.axis_index("c")
    sid = lax.axis_index("s")
    wid = sid * NC + cid
    base = wid * EPW
    sems_a = (sa0, sa1)
    sems_b = (sb0, sb1)
    sems_e = (se0, se1)

    zero16 = jnp.zeros((L,), jnp.float32)
    ones16 = jnp.full((L,), 1.0, jnp.float32)

    @pl.loop(0, N // L)
    def _(i):
        counts[pl.ds(i * L, L)] = zero16

    @pl.loop(0, D // L)
    def _(k):
        stats[0, pl.ds(k * L, L)] = zero16
        stats[1, pl.ds(k * L, L)] = zero16

    pltpu.sync_copy(row_hbm.at[wid], rowsb)
    pltpu.sync_copy(col_hbm.at[wid], colsb)

    def _issue(c, b):
        pltpu.async_copy(xa_hbm.at[rowsb.at[c]], bufa.at[b], sems_a[b])
        pltpu.async_copy(xb_hbm.at[colsb.at[c]], bufb.at[b], sems_b[b])
        pltpu.async_copy(ea_hbm.at[pl.ds(base + c * CH, CH)], bufe.at[b],
                         sems_e[b])

    def _wait(c, b):
        pltpu.make_async_copy(xa_hbm.at[rowsb.at[c]], bufa.at[b],
                              sems_a[b]).wait()
        pltpu.make_async_copy(xb_hbm.at[colsb.at[c]], bufb.at[b],
                              sems_b[b]).wait()
        pltpu.make_async_copy(ea_hbm.at[pl.ds(base + c * CH, CH)], bufe.at[b],
                              sems_e[b]).wait()

    def _compute(c, b):
        @pl.loop(0, CH // L)
        def _(j):
            idxv = rowsb[c, pl.ds(j * L, L)]
            plsc.addupdate_scatter(counts, [idxv], ones16)

        def _body(e, carry):
            out_s, out_q = [], []
            for k in range(D // L):
                sl = pl.ds(k * L, L)
                h = bufa[b, e, sl] + bufb[b, e, sl] + bufe[b, e, sl]
                bufe[b, e, sl] = h
                out_s.append(carry[k] + h)
                out_q.append(carry[(D // L) + k] + h * h)
            return tuple(out_s + out_q)

        acc = lax.fori_loop(0, CH, _body, (zero16,) * (2 * (D // L)))
        for k in range(D // L):
            plsc.addupdate(stats.at[0, pl.ds(k * L, L)], acc[k])
            plsc.addupdate(stats.at[1, pl.ds(k * L, L)], acc[(D // L) + k])
        pltpu.sync_copy(bufe.at[b], h_hbm.at[pl.ds(base + c * CH, CH)])

    _issue(0, 0)

    @pl.loop(0, NCHUNK // 2)
    def _(t):
        for b in range(2):
            c = 2 * t + b
            _wait(c, b)
            _issue(c + 1, 1 - b)
            _compute(c, b)

    _wait(NCHUNK - 1, 0)
    _compute(NCHUNK - 1, 0)

    pltpu.sync_copy(stats, stats_hbm.at[wid])
    pltpu.sync_copy(counts, cnt_hbm.at[wid])


def _stats_body(stats_ref, g_ref, b_ref, ss_ref):
    s = jnp.sum(stats_ref[...], axis=0)
    mean = s[0:1, :] * (1.0 / E)
    ex2 = s[1:2, :] * (1.0 / E)
    var = ex2 - mean * mean
    inv = lax.rsqrt(var + 1e-5)
    scale = g_ref[...] * inv
    shift = b_ref[...] - mean * scale
    ss_ref[...] = jnp.concatenate([scale, shift], axis=0)


def _stats_reduce(stats, gamma, beta):
    return pl.pallas_call(
        _stats_body,
        out_shape=jax.ShapeDtypeStruct((2, D), jnp.float32),
    )(stats, gamma.reshape(1, D), beta.reshape(1, D))


@functools.partial(
    pl.kernel,
    out_type=jax.ShapeDtypeStruct((NC, N, D), jnp.float32),
    mesh=_mesh,
    scratch_types=[
        pltpu.VMEM((NCHUNK, CH), jnp.int32),
        pltpu.VMEM((2, CH, D), jnp.float32),
        pltpu.VMEM((2, D), jnp.float32),
        pltpu.VMEM_SHARED((N, D), jnp.float32),
        pltpu.SemaphoreType.DMA,
        pltpu.SemaphoreType.DMA,
    ],
    compiler_params=_sc_params,
)
def _sc_pass2(h_hbm, row_hbm, ss_hbm, zeros_hbm,
              s_hbm,
              rowsb, buf, ss, s_sh, sh0, sh1):
    cid = lax.axis_index("c")
    sid = lax.axis_index("s")
    wid = sid * NC + cid
    base = wid * EPW
    sems = (sh0, sh1)

    pltpu.sync_copy(ss_hbm, ss)
    pltpu.sync_copy(row_hbm.at[wid], rowsb)

    @pl.when(sid < NST)
    def _():
        pltpu.sync_copy(zeros_hbm, s_sh.at[pl.ds(sid * RPT, RPT)])

    sv = [ss[0, pl.ds(k * L, L)] for k in range(D // L)]
    tv = [ss[1, pl.ds(k * L, L)] for k in range(D // L)]

    def _issue(c, b):
        pltpu.async_copy(h_hbm.at[pl.ds(base + c * CH, CH)], buf.at[b],
                         sems[b])

    def _wait(c, b):
        pltpu.make_async_copy(h_hbm.at[pl.ds(base + c * CH, CH)], buf.at[b],
                              sems[b]).wait()

    def _compute(c, b):
        @pl.loop(0, CH)
        def _(e):
            for k in range(D // L):
                sl = pl.ds(k * L, L)
                y = buf[b, e, sl] * sv[k] + tv[k]
                buf[b, e, sl] = jnp.maximum(y, 0.0)

        pltpu.sync_copy(buf.at[b], s_sh.at[rowsb.at[c]], add=True)

    _issue(0, 0)
    plsc.subcore_barrier()

    @pl.loop(0, NCHUNK // 2)
    def _(t):
        for b in range(2):
            c = 2 * t + b
            _wait(c, b)
            _issue(c + 1, 1 - b)
            _compute(c, b)

    _wait(NCHUNK - 1, 0)
    _compute(NCHUNK - 1, 0)

    plsc.subcore_barrier()

    @pl.when(sid < NST)
    def _():
        pltpu.sync_copy(
            s_sh.at[pl.ds(sid * RPT, RPT)],
            s_hbm.at[cid].at[pl.ds(sid * RPT, RPT)],
        )


def _final_body(s_ref, cnt_ref, x_ref, w_ref, b_ref, out_ref):
    s = s_ref[0] + s_ref[1]
    cnt = jnp.sum(cnt_ref[...], axis=0)[:, None]
    m = jnp.dot(s, w_ref[...], preferred_element_type=jnp.float32)
    out_ref[...] = (m + cnt * b_ref[...]) / (cnt + 1.0) + x_ref[...]


def _final(s_parts, cnts, x, w2t, b2):
    return pl.pallas_call(
        _final_body,
        out_shape=jax.ShapeDtypeStruct((N, D), jnp.float32),
    )(s_parts, cnts, x, w2t, b2.reshape(1, D))


def kernel(x, edge_index, edge_attr, W1, b1, gamma, beta, W2, b2):
    row = edge_index[0].astype(jnp.int32)
    col = edge_index[1].astype(jnp.int32)
    rows3 = row.reshape(NW, NCHUNK, CH)
    cols3 = col.reshape(NW, NCHUNK, CH)
    w1abt = jnp.concatenate([W1[:, :D].T, W1[:, D : 2 * D].T], axis=1)
    w1ct = W1[:, 2 * D :].T
    w2t = W2.T

    xa, xb = _node_proj(x, w1abt)
    ea = _edge_proj(edge_attr, w1ct, b1)
    h, stats, cnts = _sc_pass1(xa, xb, ea, rows3, cols3)
    ss = _stats_reduce(stats, gamma, beta)
    zeros = jnp.zeros((RPT, D), jnp.float32)
    s_parts = _sc_pass2(h, rows3, ss, zeros)
    return _final(s_parts, cnts, x, w2t, b2)
